# baseline (speedup 1.0000x reference)
import functools

import jax
import jax.numpy as jnp
from jax import lax
from jax.experimental import pallas as pl
from jax.experimental.pallas import tpu as pltpu

EXPERT_NUM = 64


@functools.lru_cache(maxsize=None)
def _build(t: int, k: int):
    chunks = 4
    h = t // chunks

    def body(w_hbm, ids_hbm, wout_hbm, w_v, ids_v, wout_v, sem_in0, sem_in1,
             sem_in2, sem_in3, sem_ids, sem_out):
        in_sems = (sem_in0, sem_in1, sem_in2, sem_in3)
        cp_ins = [
            pltpu.make_async_copy(
                w_hbm.at[:, pl.ds(c * h, h)],
                w_v.at[:, pl.ds(c * h, h)],
                in_sems[c],
            )
            for c in range(chunks)
        ]
        for cp in cp_ins:
            cp.start()
        tok = lax.broadcasted_iota(jnp.int32, (k, t), 1)
        kk = lax.broadcasted_iota(jnp.int32, (k, t), 0)
        ids_v[:] = (k * tok + kk) & (EXPERT_NUM - 1)
        cp_ids = pltpu.make_async_copy(ids_v, ids_hbm, sem_ids)
        cp_ids.start()
        cps = []
        for c in range(chunks):
            sl = pl.ds(c * h, h)
            cp_ins[c].wait()
            x = w_v[:, sl]
            wout_v[:, sl] = x / jnp.maximum(x[0:1, :] + x[1:2, :], 1e-9)
            cp = pltpu.make_async_copy(
                wout_v.at[:, sl], wout_hbm.at[:, sl], sem_out
            )
            cp.start()
            cps.append(cp)
        cp_ids.wait()
        for cp in cps:
            cp.wait()

    return pl.pallas_call(
        body,
        in_specs=[pl.BlockSpec(memory_space=pltpu.MemorySpace.HBM)],
        out_specs=[
            pl.BlockSpec(memory_space=pltpu.MemorySpace.HBM),
            pl.BlockSpec(memory_space=pltpu.MemorySpace.HBM),
        ],
        out_shape=[
            jax.ShapeDtypeStruct((k, t), jnp.int32),
            jax.ShapeDtypeStruct((k, t), jnp.float32),
        ],
        scratch_shapes=[
            pltpu.VMEM((k, t), jnp.float32),
            pltpu.VMEM((k, t), jnp.int32),
            pltpu.VMEM((k, t), jnp.float32),
            pltpu.SemaphoreType.DMA,
            pltpu.SemaphoreType.DMA,
            pltpu.SemaphoreType.DMA,
            pltpu.SemaphoreType.DMA,
            pltpu.SemaphoreType.DMA,
            pltpu.SemaphoreType.DMA,
        ],
    )


def kernel(topk_ids, topk_weights):
    t, k = topk_ids.shape
    ids_t, wout_t = _build(t, k)(topk_weights.T)
    return ids_t.T, wout_t.T


# speedup vs baseline: 1.0096x; 1.0096x over previous
import functools

import jax
import jax.numpy as jnp
from jax import lax
from jax.experimental import pallas as pl
from jax.experimental.pallas import tpu as pltpu

EXPERT_NUM = 64


@functools.lru_cache(maxsize=None)
def _build(t: int, k: int):
    h = t // 2

    def body(w_hbm, ids_hbm, wout_hbm, w_v, ids_v, wout_v, sem_in0, sem_in1,
             sem_ids, sem_out):
        cp_ins = [
            pltpu.make_async_copy(
                w_hbm.at[:, pl.ds(half * h, h)],
                w_v.at[:, pl.ds(half * h, h)],
                sem,
            )
            for half, sem in ((0, sem_in0), (1, sem_in1))
        ]
        for cp in cp_ins:
            cp.start()
        tok = lax.broadcasted_iota(jnp.int32, (k, t), 1)
        kk = lax.broadcasted_iota(jnp.int32, (k, t), 0)
        ids_v[:] = (k * tok + kk) & (EXPERT_NUM - 1)
        cp_ids = pltpu.make_async_copy(ids_v, ids_hbm, sem_ids)
        cp_ids.start()
        cps = []
        for half in range(2):
            sl = pl.ds(half * h, h)
            cp_ins[half].wait()
            x = w_v[:, sl]
            wout_v[:, sl] = x / jnp.maximum(x[0:1, :] + x[1:2, :], 1e-9)
            cp = pltpu.make_async_copy(
                wout_v.at[:, sl], wout_hbm.at[:, sl], sem_out
            )
            cp.start()
            cps.append(cp)
        cp_ids.wait()
        for cp in cps:
            cp.wait()

    return pl.pallas_call(
        body,
        in_specs=[pl.BlockSpec(memory_space=pltpu.MemorySpace.HBM)],
        out_specs=[
            pl.BlockSpec(memory_space=pltpu.MemorySpace.HBM),
            pl.BlockSpec(memory_space=pltpu.MemorySpace.HBM),
        ],
        out_shape=[
            jax.ShapeDtypeStruct((k, t), jnp.int32),
            jax.ShapeDtypeStruct((k, t), jnp.float32),
        ],
        scratch_shapes=[
            pltpu.VMEM((k, t), jnp.float32),
            pltpu.VMEM((k, t), jnp.int32),
            pltpu.VMEM((k, t), jnp.float32),
            pltpu.SemaphoreType.DMA,
            pltpu.SemaphoreType.DMA,
            pltpu.SemaphoreType.DMA,
            pltpu.SemaphoreType.DMA,
        ],
    )


def kernel(topk_ids, topk_weights):
    t, k = topk_ids.shape
    ids_t, wout_t = _build(t, k)(topk_weights.T)
    return ids_t.T, wout_t.T
